# Initial kernel scaffold; baseline (speedup 1.0000x reference)
import jax
import jax.numpy as jnp
from jax.experimental import pallas as pl


def kernel(x, table, W, b):
    raise NotImplementedError("write your pallas kernel here")


# speedup vs baseline: 15.4278x; 15.4278x over previous
import functools

import jax
import jax.numpy as jnp
from jax import lax
from jax.experimental import pallas as pl
from jax.experimental.pallas import tpu as pltpu
from jax.experimental.pallas import tpu_sc as plsc

D = 64
HALF = 100000
NEMB = 2 * HALF

NC, NS = 2, 16
NW = NC * NS

PROJ_BLK = 10000

CHUNK_TOK = 128
CHUNK_IDX = 2 * CHUNK_TOK
GATHER_IDX = 128


def _proj_body(t_ref, w_ref, b_ref, p_ref):
    p_ref[...] = (
        jnp.dot(t_ref[...], w_ref[...], preferred_element_type=jnp.float32)
        + 0.5 * b_ref[...]
    )


def _project_table(table, Wcat, b2):
    return pl.pallas_call(
        _proj_body,
        grid=(NEMB // PROJ_BLK,),
        in_specs=[
            pl.BlockSpec((PROJ_BLK, D), lambda i: (i, 0)),
            pl.BlockSpec((D, 2 * D), lambda i: (0, 0)),
            pl.BlockSpec((1, 2 * D), lambda i: (0, 0)),
        ],
        out_specs=pl.BlockSpec((PROJ_BLK, 2 * D), lambda i: (i, 0)),
        out_shape=jax.ShapeDtypeStruct((NEMB, 2 * D), jnp.float32),
    )(table, Wcat, b2)


def _sc_body(n_tok, xi_hbm, p_hbm, out_hbm,
             idx_v, rows0, rows1, out0, out1, g0, g1, o0, o1):
    tok_per_w = n_tok // NW
    idx_per_w = 2 * tok_per_w
    n_chunks = tok_per_w // CHUNK_TOK
    wid = lax.axis_index("s") * NC + lax.axis_index("c")
    rows = (rows0, rows1)
    outs = (out0, out1)
    gsems = (g0, g1)
    osems = (o0, o1)

    pltpu.sync_copy(xi_hbm.at[pl.ds(wid * idx_per_w, idx_per_w)], idx_v)
    offs = (lax.iota(jnp.int32, 16) % 2) * (NEMB + 1)

    @plsc.parallel_loop(0, idx_per_w // 16, 1, unroll=8)
    def _(t):
        v = idx_v[pl.ds(t * 16, 16)]
        idx_v[pl.ds(t * 16, 16)] = v + v + offs

    def fire(c, buf):
        for g in range(CHUNK_IDX // GATHER_IDX):
            pltpu.async_copy(
                p_hbm.at[idx_v.at[pl.ds(c * CHUNK_IDX + g * GATHER_IDX,
                                        GATHER_IDX)]],
                rows[buf].at[pl.ds(g * GATHER_IDX, GATHER_IDX)],
                gsems[buf],
            )

    def drain_gather(buf):
        for g in range(CHUNK_IDX // GATHER_IDX):
            pltpu.make_async_copy(
                p_hbm.at[pl.ds(0, GATHER_IDX)],
                rows[buf].at[pl.ds(g * GATHER_IDX, GATHER_IDX)],
                gsems[buf],
            ).wait()

    def drain_out(buf):
        pltpu.make_async_copy(
            out_hbm.at[pl.ds(0, CHUNK_TOK)],
            outs[buf],
            osems[buf],
        ).wait()

    def compute(buf, c):
        @plsc.parallel_loop(0, CHUNK_TOK, 1, unroll=2)
        def _(j):
            for v in range(D // 16):
                s = v * 16
                outs[buf][j, pl.ds(s, 16)] = (
                    rows[buf][2 * j, pl.ds(s, 16)]
                    + rows[buf][2 * j + 1, pl.ds(s, 16)]
                )
        pltpu.async_copy(
            outs[buf],
            out_hbm.at[pl.ds(wid * tok_per_w + c * CHUNK_TOK, CHUNK_TOK)],
            osems[buf],
        )

    fire(0, 0)

    def pair(k, carry):
        c0 = 2 * k
        fire(c0 + 1, 1)
        drain_gather(0)

        @pl.when(c0 >= 2)
        def _():
            drain_out(0)

        compute(0, c0)

        @pl.when(c0 + 2 < n_chunks)
        def _():
            fire(c0 + 2, 0)

        drain_gather(1)

        @pl.when(c0 >= 2)
        def _():
            drain_out(1)

        compute(1, c0 + 1)
        return carry

    lax.fori_loop(0, n_chunks // 2, pair, 0)
    drain_out(0)
    drain_out(1)


def _sc_lookup(xi, P, n_tok):
    mesh = plsc.VectorSubcoreMesh(core_axis_name="c", subcore_axis_name="s")
    idx_per_w = 2 * (n_tok // NW)
    f = pl.kernel(
        functools.partial(_sc_body, n_tok),
        out_type=jax.ShapeDtypeStruct((n_tok, D), jnp.float32),
        mesh=mesh,
        compiler_params=pltpu.CompilerParams(use_tc_tiling_on_sc=False),
        scratch_types=[
            pltpu.VMEM((idx_per_w,), jnp.int32),
            pltpu.VMEM((CHUNK_IDX, D), jnp.float32),
            pltpu.VMEM((CHUNK_IDX, D), jnp.float32),
            pltpu.VMEM((CHUNK_TOK, D), jnp.float32),
            pltpu.VMEM((CHUNK_TOK, D), jnp.float32),
            pltpu.SemaphoreType.DMA,
            pltpu.SemaphoreType.DMA,
            pltpu.SemaphoreType.DMA,
            pltpu.SemaphoreType.DMA,
        ],
    )
    return f(xi, P)


def kernel(x, table, W, b):
    B, S, T, F = x.shape
    n_tok = B * S * T
    xflat = x.reshape(-1).astype(jnp.int32)
    Wcat = W.T.reshape(F, D, D).transpose(1, 0, 2).reshape(D, 2 * D)
    b2 = jnp.concatenate([b, b]).reshape(1, 2 * D)
    P2 = _project_table(table, Wcat, b2)
    out = _sc_lookup(xflat, P2.reshape(2 * NEMB, D), n_tok)
    return out.reshape(B, S, T, D)


# speedup vs baseline: 16.0287x; 1.0389x over previous
import functools

import jax
import jax.numpy as jnp
from jax import lax
from jax.experimental import pallas as pl
from jax.experimental.pallas import tpu as pltpu
from jax.experimental.pallas import tpu_sc as plsc

D = 64
HALF = 100000
NEMB = 2 * HALF

NC, NS = 2, 16
NW = NC * NS

PROJ_BLK = 12800

SEQ = 50
BW = 128
GATHER_IDX = 128


def _proj_body(tt_ref, w_ref, b_ref, p_ref):
    p_ref[...] = (
        lax.dot_general(
            tt_ref[...], w_ref[...],
            dimension_numbers=(((0,), (0,)), ((), ())),
            preferred_element_type=jnp.float32,
        )
        + 0.5 * b_ref[...]
    )


def _project_table(tableT, Wcat, b2):
    return pl.pallas_call(
        _proj_body,
        grid=(pl.cdiv(NEMB, PROJ_BLK),),
        in_specs=[
            pl.BlockSpec((D, PROJ_BLK), lambda i: (0, i)),
            pl.BlockSpec((D, 2 * D), lambda i: (0, 0)),
            pl.BlockSpec((1, 2 * D), lambda i: (0, 0)),
        ],
        out_specs=pl.BlockSpec((PROJ_BLK, 2 * D), lambda i: (i, 0)),
        out_shape=jax.ShapeDtypeStruct((NEMB, 2 * D), jnp.float32),
    )(tableT, Wcat, b2)


def _sc_body(xp_hbm, p_hbm, out_hbm,
             idx2, rows0, rows1, out0, out1, g0, g1, o0, o1):
    wid = lax.axis_index("s") * NC + lax.axis_index("c")
    b0 = wid * BW
    rows = (rows0, rows1)
    outs = (out0, out1)
    gsems = (g0, g1)
    osems = (o0, o1)

    pltpu.sync_copy(xp_hbm.at[:, wid, :], idx2)

    def xl8(s, carry):
        for g in range(2 * BW // 16):
            off = 0 if g < BW // 16 else NEMB + 1
            v = idx2[s, pl.ds(g * 16, 16)]
            idx2[s, pl.ds(g * 16, 16)] = v + v + off
        return carry

    lax.fori_loop(0, SEQ, xl8, 0)

    def fire(s, buf):
        for g in range(2):
            pltpu.async_copy(
                p_hbm.at[idx2.at[s, pl.ds(g * GATHER_IDX, GATHER_IDX)]],
                rows[buf].at[pl.ds(g * GATHER_IDX, GATHER_IDX)],
                gsems[buf],
            )

    def drain_gather(buf):
        for g in range(2):
            pltpu.make_async_copy(
                p_hbm.at[pl.ds(0, GATHER_IDX)],
                rows[buf].at[pl.ds(g * GATHER_IDX, GATHER_IDX)],
                gsems[buf],
            ).wait()

    def drain_out(buf):
        pltpu.make_async_copy(
            out_hbm.at[pl.ds(0, BW), 0],
            outs[buf],
            osems[buf],
        ).wait()

    def compute(s, buf):
        @plsc.parallel_loop(0, BW, 1, unroll=2)
        def _(j):
            for v in range(D // 16):
                c = v * 16
                outs[buf][j, 0, pl.ds(c, 16)] = (
                    rows[buf][j, pl.ds(c, 16)]
                    + rows[buf][BW + j, pl.ds(c, 16)]
                )
        pltpu.async_copy(
            outs[buf],
            out_hbm.at[pl.ds(b0, BW), s],
            osems[buf],
        )

    fire(0, 0)

    def pair(k, carry):
        s0 = 2 * k
        fire(s0 + 1, 1)
        drain_gather(0)

        @pl.when(s0 >= 2)
        def _():
            drain_out(0)

        compute(s0, 0)

        @pl.when(s0 + 2 < SEQ)
        def _():
            fire(s0 + 2, 0)

        drain_gather(1)

        @pl.when(s0 >= 2)
        def _():
            drain_out(1)

        compute(s0 + 1, 1)
        return carry

    lax.fori_loop(0, SEQ // 2, pair, 0)
    drain_out(0)
    drain_out(1)


def _sc_lookup(xp, Pl, B):
    mesh = plsc.VectorSubcoreMesh(core_axis_name="c", subcore_axis_name="s")
    f = pl.kernel(
        _sc_body,
        out_type=jax.ShapeDtypeStruct((B, SEQ, 1, D), jnp.float32),
        mesh=mesh,
        compiler_params=pltpu.CompilerParams(use_tc_tiling_on_sc=False),
        scratch_types=[
            pltpu.VMEM((SEQ, 2 * BW), jnp.int32),
            pltpu.VMEM((2 * BW, D), jnp.float32),
            pltpu.VMEM((2 * BW, D), jnp.float32),
            pltpu.VMEM((BW, 1, D), jnp.float32),
            pltpu.VMEM((BW, 1, D), jnp.float32),
            pltpu.SemaphoreType.DMA,
            pltpu.SemaphoreType.DMA,
            pltpu.SemaphoreType.DMA,
            pltpu.SemaphoreType.DMA,
        ],
    )
    return f(xp, Pl)


def kernel(x, table, W, b):
    B, S, T, F = x.shape
    xp = (
        x.reshape(B, S, F)
        .transpose(1, 2, 0)
        .reshape(S, F, B // BW, BW)
        .transpose(0, 2, 1, 3)
        .reshape(S, B // BW, F * BW)
        .astype(jnp.int32)
    )
    Wcat = W.T.reshape(F, D, D).transpose(1, 0, 2).reshape(D, 2 * D)
    b2 = jnp.concatenate([b, b]).reshape(1, 2 * D)
    P2 = _project_table(jnp.transpose(table), Wcat, b2)
    return _sc_lookup(xp, P2.reshape(2 * NEMB, D), B)
